# baseline (speedup 1.0000x reference)
import functools

import jax
import jax.numpy as jnp
from jax import lax
from jax.experimental import pallas as pl
from jax.experimental.pallas import tpu as pltpu
from jax.experimental.pallas import tpu_sc as plsc

B = 1024
N = 20000
N_DN = 1300
N_MBON = 96
NC = 2
NS = 16
NW = NC * NS
BLK = 8
DN_BLKS = (N_DN + BLK - 1) // BLK
DN_PAD = DN_BLKS * BLK
MB_BLKS = N_MBON // BLK
TOT_BLKS = DN_BLKS + MB_BLKS
IDX_PAD = TOT_BLKS * BLK
MAX_BLKS_PER_W = (TOT_BLKS + NW - 1) // NW
NBUF = 3


def _sc_body(vt_hbm, cidx_hbm, dnt_hbm, mbt_hbm,
             cidx_v, g0, g1, g2, sg0, sg1, sg2, so0, so1, so2):
    wid = lax.axis_index("s") * NC + lax.axis_index("c")
    gb = (g0, g1, g2)
    sg = (sg0, sg1, sg2)
    so = (so0, so1, so2)

    pltpu.sync_copy(cidx_hbm, cidx_v)

    def gather_copy(k, blk):
        return pltpu.make_async_copy(
            vt_hbm.at[cidx_v.at[pl.ds(blk * BLK, BLK)]],
            gb[k % NBUF], sg[k % NBUF])

    def dn_copy(k, blk):
        return pltpu.make_async_copy(
            gb[k % NBUF], dnt_hbm.at[pl.ds(blk * BLK, BLK)], so[k % NBUF])

    def mb_copy(k, blk):
        return pltpu.make_async_copy(
            gb[k % NBUF], mbt_hbm.at[pl.ds((blk - DN_BLKS) * BLK, BLK)],
            so[k % NBUF])

    def issue_gather(k):
        blk = wid + NW * k

        @pl.when(blk < TOT_BLKS)
        def _():
            gather_copy(k, blk).start()

    def wait_gather(k):
        blk = wid + NW * k

        @pl.when(blk < TOT_BLKS)
        def _():
            gather_copy(k, blk).wait()

    def issue_out(k):
        blk = wid + NW * k

        @pl.when(blk < DN_BLKS)
        def _():
            dn_copy(k, blk).start()

        @pl.when((blk >= DN_BLKS) & (blk < TOT_BLKS))
        def _():
            mb_copy(k, blk).start()

    def wait_out(k):
        blk = wid + NW * k

        @pl.when(blk < DN_BLKS)
        def _():
            dn_copy(k, blk).wait()

        @pl.when((blk >= DN_BLKS) & (blk < TOT_BLKS))
        def _():
            mb_copy(k, blk).wait()

    for k in range(min(NBUF, MAX_BLKS_PER_W)):
        issue_gather(k)
    for k in range(MAX_BLKS_PER_W):
        if k >= NBUF:
            wait_out(k - NBUF)
            issue_gather(k)
        wait_gather(k)
        issue_out(k)
    for k in range(max(0, MAX_BLKS_PER_W - NBUF), MAX_BLKS_PER_W):
        wait_out(k)


@jax.jit
def kernel(v, dn_id, mbon_id):
    vt = jnp.swapaxes(v, 0, 1)
    cidx = jnp.concatenate(
        [dn_id.astype(jnp.int32),
         jnp.zeros((DN_PAD - N_DN,), jnp.int32),
         mbon_id.astype(jnp.int32)])

    mesh = plsc.VectorSubcoreMesh(core_axis_name="c", subcore_axis_name="s")
    run = pl.kernel(
        _sc_body,
        mesh=mesh,
        compiler_params=pltpu.CompilerParams(needs_layout_passes=False,
                                             use_tc_tiling_on_sc=True,
                                             skip_device_barrier=True,
                                             disable_bounds_checks=True,
                                             disable_semaphore_checks=True),
        out_type=(jax.ShapeDtypeStruct((DN_PAD, B), jnp.float32),
                  jax.ShapeDtypeStruct((N_MBON, B), jnp.float32)),
        scratch_types=(
            [pltpu.VMEM((IDX_PAD,), jnp.int32)]
            + [pltpu.VMEM((BLK, B), jnp.float32) for _ in range(NBUF)]
            + [pltpu.SemaphoreType.DMA for _ in range(2 * NBUF)]
        ),
    )
    dnt, mbt = run(vt, cidx)
    return (jnp.swapaxes(dnt[:N_DN], 0, 1), jnp.swapaxes(mbt, 0, 1))


# speedup vs baseline: 1.2787x; 1.2787x over previous
import functools

import jax
import jax.numpy as jnp
from jax import lax
from jax.experimental import pallas as pl
from jax.experimental.pallas import tpu as pltpu
from jax.experimental.pallas import tpu_sc as plsc

B = 1024
N = 20000
N_DN = 1300
N_MBON = 96
NC = 2
NS = 16
NW = NC * NS
BLK = 16
DN_BLKS = (N_DN + BLK - 1) // BLK
DN_TAIL = N_DN - (DN_BLKS - 1) * BLK
MB_BLKS = N_MBON // BLK
TOT_BLKS = DN_BLKS + MB_BLKS
IDX_PAD = TOT_BLKS * BLK
MAX_BLKS_PER_W = (TOT_BLKS + NW - 1) // NW
NBUF = 3


def _sc_body(vt_hbm, cidx_hbm, dnt_hbm, mbt_hbm,
             cidx_v, g0, g1, g2, sg0, sg1, sg2, so0, so1, so2):
    wid = lax.axis_index("s") * NC + lax.axis_index("c")
    gb = (g0, g1, g2)
    sg = (sg0, sg1, sg2)
    so = (so0, so1, so2)

    pltpu.sync_copy(cidx_hbm, cidx_v)

    def gather_copy(k, blk):
        return pltpu.make_async_copy(
            vt_hbm.at[cidx_v.at[pl.ds(blk * BLK, BLK)]],
            gb[k % NBUF], sg[k % NBUF])

    def full_dn_copy(k, blk):
        return pltpu.make_async_copy(
            gb[k % NBUF], dnt_hbm.at[pl.ds(blk * BLK, BLK)], so[k % NBUF])

    def part_dn_copy(k):
        return pltpu.make_async_copy(
            gb[k % NBUF].at[pl.ds(0, DN_TAIL)],
            dnt_hbm.at[pl.ds((DN_BLKS - 1) * BLK, DN_TAIL)], so[k % NBUF])

    def mb_copy(k, blk):
        return pltpu.make_async_copy(
            gb[k % NBUF], mbt_hbm.at[pl.ds((blk - DN_BLKS) * BLK, BLK)],
            so[k % NBUF])

    def issue_gather(k):
        blk = wid + NW * k

        @pl.when(blk < TOT_BLKS)
        def _():
            gather_copy(k, blk).start()

    def wait_gather(k):
        blk = wid + NW * k

        @pl.when(blk < TOT_BLKS)
        def _():
            gather_copy(k, blk).wait()

    def each_out(k, fn):
        blk = wid + NW * k

        @pl.when(blk < DN_BLKS - 1)
        def _():
            fn(full_dn_copy(k, blk))

        @pl.when(blk == DN_BLKS - 1)
        def _():
            fn(part_dn_copy(k))

        @pl.when((blk >= DN_BLKS) & (blk < TOT_BLKS))
        def _():
            fn(mb_copy(k, blk))

    for k in range(min(NBUF, MAX_BLKS_PER_W)):
        issue_gather(k)
    for k in range(MAX_BLKS_PER_W):
        if k >= NBUF:
            each_out(k - NBUF, lambda c: c.wait())
            issue_gather(k)
        wait_gather(k)
        each_out(k, lambda c: c.start())
    for k in range(max(0, MAX_BLKS_PER_W - NBUF), MAX_BLKS_PER_W):
        each_out(k, lambda c: c.wait())


@jax.jit
def kernel(v, dn_id, mbon_id):
    vt = jnp.swapaxes(v, 0, 1)
    cidx = jnp.concatenate(
        [dn_id.astype(jnp.int32),
         jnp.zeros(((DN_BLKS - 1) * BLK + BLK - N_DN,), jnp.int32),
         mbon_id.astype(jnp.int32)])

    mesh = plsc.VectorSubcoreMesh(core_axis_name="c", subcore_axis_name="s")
    run = pl.kernel(
        _sc_body,
        mesh=mesh,
        compiler_params=pltpu.CompilerParams(needs_layout_passes=False,
                                             use_tc_tiling_on_sc=True,
                                             skip_device_barrier=True,
                                             disable_bounds_checks=True,
                                             disable_semaphore_checks=True),
        out_type=(jax.ShapeDtypeStruct((N_DN, B), jnp.float32),
                  jax.ShapeDtypeStruct((N_MBON, B), jnp.float32)),
        scratch_types=(
            [pltpu.VMEM((IDX_PAD,), jnp.int32)]
            + [pltpu.VMEM((BLK, B), jnp.float32) for _ in range(NBUF)]
            + [pltpu.SemaphoreType.DMA for _ in range(2 * NBUF)]
        ),
    )
    dnt, mbt = run(vt, cidx)
    return jnp.swapaxes(dnt, 0, 1), jnp.swapaxes(mbt, 0, 1)
